# baseline (speedup 1.0000x reference)
import jax
import jax.numpy as jnp
from jax import lax
from jax.experimental import pallas as pl
from jax.experimental.pallas import tpu as pltpu
from jax.experimental.pallas import tpu_sc as plsc

N = 10000
NP = 10240
E = 160000
D_IN = 256
D_H = 200
DP = 256
HALF = 128
M = 80000
MEAN_RATING = 3.5

NC = 2
NS = 16
NW = NC * NS
CHUNK = 128
NCH_E = 1280
TCH = NCH_E // NS
M_PAD = 81920
NCH_M = M_PAD // CHUNK
DCH = NCH_M // NS
ROWS_PER_TILE = NP // NS
ZROWS = 16

_MESH = plsc.VectorSubcoreMesh(core_axis_name="c", subcore_axis_name="s")


def _pad_w(w):
    return jnp.pad(w, ((0, 0), (0, DP - D_H)))


def _pad_idx(idx1d, nch):
    n_pad = NCH_E * CHUNK - idx1d.shape[0]
    pad = N + (jnp.arange(n_pad, dtype=jnp.int32) % 240)
    return jnp.concatenate([idx1d, pad]).reshape(NCH_E, CHUNK)


_RB = 1024


def _mm1_body(x_ref, w_ref, xfl_ref, xfh_ref, xsl_ref, xsh_ref,
              xrl_ref, xrh_ref):
    prod = jnp.dot(x_ref[...], w_ref[...], preferred_element_type=jnp.float32)
    xfl_ref[...] = prod[:, :HALF]
    xfh_ref[...] = prod[:, HALF:DP]
    xsl_ref[...] = prod[:, DP:DP + HALF]
    xsh_ref[...] = prod[:, DP + HALF:2 * DP]
    xrl_ref[...] = prod[:, 2 * DP:2 * DP + HALF]
    xrh_ref[...] = prod[:, 2 * DP + HALF:]


def _mm1(x, w1):
    return pl.pallas_call(
        _mm1_body,
        grid=(NP // _RB,),
        in_specs=[
            pl.BlockSpec((_RB, D_IN), lambda i: (i, 0)),
            pl.BlockSpec((D_IN, 3 * DP), lambda i: (0, 0)),
        ],
        out_specs=[pl.BlockSpec((_RB, HALF), lambda i: (i, 0))] * 6,
        out_shape=[jax.ShapeDtypeStruct((NP, HALF), jnp.float32)] * 6,
    )(x, w1)


def _mm2_body(pl_ref, ph_ref, b_ref, w_ref,
              hfl_ref, hfh_ref, hsl_ref, hsh_ref, hrl_ref, hrh_ref):
    agg = jnp.concatenate([pl_ref[...], ph_ref[...]], axis=1)
    h = jax.nn.relu(agg + b_ref[...])
    prod = jnp.dot(h, w_ref[...], preferred_element_type=jnp.float32)
    hfl_ref[...] = prod[:, :HALF]
    hfh_ref[...] = prod[:, HALF:DP]
    hsl_ref[...] = prod[:, DP:DP + HALF]
    hsh_ref[...] = prod[:, DP + HALF:2 * DP]
    hrl_ref[...] = prod[:, 2 * DP:2 * DP + HALF]
    hrh_ref[...] = prod[:, 2 * DP + HALF:]


def _mm2(p_lo, p_hi, b1, w2):
    return pl.pallas_call(
        _mm2_body,
        grid=(NP // _RB,),
        in_specs=[
            pl.BlockSpec((_RB, HALF), lambda i: (i, 0)),
            pl.BlockSpec((_RB, HALF), lambda i: (i, 0)),
            pl.BlockSpec((1, DP), lambda i: (0, 0)),
            pl.BlockSpec((DP, 3 * DP), lambda i: (0, 0)),
        ],
        out_specs=[pl.BlockSpec((_RB, HALF), lambda i: (i, 0))] * 6,
        out_shape=[jax.ShapeDtypeStruct((NP, HALF), jnp.float32)] * 6,
    )(p_lo, p_hi, b1, w2)


def _mm3_body(rl_ref, rh_ref, b_ref, q_ref,
              gl_ref, gh_ref, ql_ref, qh_ref):
    agg = jnp.concatenate([rl_ref[...], rh_ref[...]], axis=1)
    g = agg + b_ref[...]
    gl_ref[...] = g[:, :HALF]
    gh_ref[...] = g[:, HALF:]
    gq = jnp.dot(g, q_ref[...], preferred_element_type=jnp.float32)
    ql_ref[...] = gq[:, :HALF]
    qh_ref[...] = gq[:, HALF:]


def _mm3(r_lo, r_hi, b2, q):
    return pl.pallas_call(
        _mm3_body,
        grid=(NP // _RB,),
        in_specs=[
            pl.BlockSpec((_RB, HALF), lambda i: (i, 0)),
            pl.BlockSpec((_RB, HALF), lambda i: (i, 0)),
            pl.BlockSpec((1, DP), lambda i: (0, 0)),
            pl.BlockSpec((DP, DP), lambda i: (0, 0)),
        ],
        out_specs=[pl.BlockSpec((_RB, HALF), lambda i: (i, 0))] * 4,
        out_shape=[jax.ShapeDtypeStruct((NP, HALF), jnp.float32)] * 4,
    )(r_lo, r_hi, b2, q)


def _fin_body(lo_ref, hi_ref, o_ref):
    o_ref[...] = lo_ref[...] + hi_ref[...] + MEAN_RATING


def _fin(lo, hi):
    return pl.pallas_call(
        _fin_body,
        in_specs=[
            pl.BlockSpec((NCH_M, CHUNK), lambda: (0, 0)),
            pl.BlockSpec((NCH_M, CHUNK), lambda: (0, 0)),
        ],
        out_specs=pl.BlockSpec((NCH_M, CHUNK), lambda: (0, 0)),
        out_shape=jax.ShapeDtypeStruct((NCH_M, CHUNK), jnp.float32),
    )(lo, hi)


BCH = 40


def _segsum_body(xs_lo, xs_hi, xr_lo, xr_hi, xf_lo, xf_hi, sim_src,
                 sim_dst, rat_src, rat_dst, plo_hbm, phi_hbm,
                 acc, idx_s, idx_d, r0, r1, g0, g1, s0, s1):
    c = lax.axis_index("c")
    s = lax.axis_index("s")
    rows = [r0, r1]
    semg = [g0, g1]
    sems = [s0, s1]

    base_row = s * ROWS_PER_TILE

    @pl.when(c == 0)
    def _():
        pltpu.sync_copy(xf_lo.at[pl.ds(base_row, ROWS_PER_TILE)],
                        acc.at[pl.ds(base_row, ROWS_PER_TILE)])

    @pl.when(c == 1)
    def _():
        pltpu.sync_copy(xf_hi.at[pl.ds(base_row, ROWS_PER_TILE)],
                        acc.at[pl.ds(base_row, ROWS_PER_TILE)])

    plsc.subcore_barrier()

    def process(table_hbm, src2d, dst2d, core_id):
        @pl.when(c == core_id)
        def _():
            def gather(t, k):
                pltpu.async_copy(table_hbm.at[idx_s.at[t]], rows[k], semg[k])

            def scat(t, k):
                pltpu.async_copy(rows[k], acc.at[idx_d.at[t]], sems[k],
                                 add=True)

            def wait_g(k):
                pltpu.make_async_copy(table_hbm.at[pl.ds(0, CHUNK)],
                                      rows[k], semg[k]).wait()

            def wait_s(k):
                pltpu.make_async_copy(rows[k], acc.at[pl.ds(0, CHUNK)],
                                      sems[k]).wait()

            def blk(b, carry):
                blk_row = s * TCH + b * BCH
                pltpu.sync_copy(src2d.at[pl.ds(blk_row, BCH)], idx_s)
                pltpu.sync_copy(dst2d.at[pl.ds(blk_row, BCH)], idx_d)
                gather(0, 0)

                def body(i, carry2):
                    for k in (0, 1):
                        t = 2 * i + k
                        wait_g(k)
                        if k == 0:
                            @pl.when(i > 0)
                            def _():
                                wait_s(1)

                            gather(t + 1, 1)
                        else:
                            wait_s(0)

                            @pl.when(i < BCH // 2 - 1)
                            def _():
                                gather(t + 1, 0)

                        scat(t, k)
                    return carry2

                lax.fori_loop(0, BCH // 2, body, 0)
                wait_s(1)
                return carry

            lax.fori_loop(0, TCH // BCH, blk, 0)

    process(xs_lo, sim_src, sim_dst, 0)
    process(xs_hi, sim_src, sim_dst, 1)
    process(xr_lo, rat_src, rat_dst, 0)
    process(xr_hi, rat_src, rat_dst, 1)
    plsc.subcore_barrier()

    def dump(out_hbm, core_id):
        @pl.when(c == core_id)
        def _():
            pltpu.sync_copy(acc.at[pl.ds(base_row, ROWS_PER_TILE)],
                            out_hbm.at[pl.ds(base_row, ROWS_PER_TILE)])

    dump(plo_hbm, 0)
    dump(phi_hbm, 1)


_segsum = pl.kernel(
    _segsum_body,
    out_type=[jax.ShapeDtypeStruct((NP, HALF), jnp.float32)] * 2,
    mesh=_MESH,
    scratch_types=[
        pltpu.VMEM_SHARED((NP, HALF), jnp.float32),
        pltpu.VMEM((BCH, CHUNK), jnp.int32),
        pltpu.VMEM((BCH, CHUNK), jnp.int32),
    ]
    + [pltpu.VMEM((CHUNK, HALF), jnp.float32)] * 2
    + [pltpu.SemaphoreType.DMA] * 4,
)


def _decode_body(g_lo, g_hi, gq_lo, gq_hi, rat_src, rat_dst, mask2d,
                 lo_hbm, hi_hbm,
                 mvec, e1a, e1b, e2a, e2b, ra0, ra1, rb0, rb1, out_buf,
                 sm, se0, se1, sr0, sr1):
    c = lax.axis_index("c")
    s = lax.axis_index("s")
    e1 = [e1a, e1b]
    e2 = [e2a, e2b]
    ra = [ra0, ra1]
    rb = [rb0, rb1]
    seme = [se0, se1]
    semr = [sr0, sr1]

    def process(gt, qt, out_hbm, core_id):
        @pl.when(c == core_id)
        def _():
            pltpu.sync_copy(mask2d.at[pl.ds(s * DCH, DCH)], mvec)

            def resolve(t, k):
                pltpu.async_copy(rat_src.at[mvec.at[t]], e1[k], seme[k])
                pltpu.async_copy(rat_dst.at[mvec.at[t]], e2[k], seme[k])

            def wait_e(k):
                pltpu.make_async_copy(rat_src.at[pl.ds(0, CHUNK)],
                                      e1[k], seme[k]).wait()
                pltpu.make_async_copy(rat_src.at[pl.ds(0, CHUNK)],
                                      e2[k], seme[k]).wait()

            def rowgather(t, k):
                pltpu.async_copy(gt.at[e1[k]], ra[k], semr[k])
                pltpu.async_copy(qt.at[e2[k]], rb[k], semr[k])

            def wait_r(k):
                pltpu.make_async_copy(gt.at[pl.ds(0, CHUNK)],
                                      ra[k], semr[k]).wait()
                pltpu.make_async_copy(gt.at[pl.ds(0, CHUNK)],
                                      rb[k], semr[k]).wait()

            def compute(t, k):
                lanes = lax.iota(jnp.int32, 16)
                msk = {h: (lanes & (2 * h - 1)) < h for h in (8, 4, 2, 1)}

                def gbody(gi, carry):
                    vecs = []
                    for l in range(16):
                        e = gi * 16 + l
                        prods = [ra[k][e, pl.ds(16 * kk, 16)]
                                 * rb[k][e, pl.ds(16 * kk, 16)]
                                 for kk in range(HALF // 16)]
                        while len(prods) > 1:
                            prods = [prods[2 * i] + prods[2 * i + 1]
                                     for i in range(len(prods) // 2)]
                        vecs.append(prods[0])
                    while len(vecs) > 1:
                        half = len(vecs) // 2
                        h = half
                        nxt = []
                        for i in range(half):
                            a, b = vecs[i], vecs[i + half]
                            sm[2 * i, pl.ds(0, 16)] = a
                            da = sm[2 * i, pl.ds(h, 16)]
                            sm[2 * i + 1, pl.ds(h, 16)] = b
                            ub = sm[2 * i + 1, pl.ds(0, 16)]
                            nxt.append(jnp.where(msk[h], a + da, b + ub))
                        vecs = nxt
                    out_buf[t, gi, :] = vecs[0]
                    return carry

                lax.fori_loop(0, CHUNK // 16, gbody, 0)

            resolve(0, 0)
            wait_e(0)
            rowgather(0, 0)
            resolve(1, 1)

            def body(i, carry):
                for k in (0, 1):
                    t = 2 * i + k
                    y = 1 - k
                    wait_r(k)
                    if k == 0:
                        wait_e(y)
                        rowgather(t + 1, y)

                        @pl.when(i < DCH // 2 - 1)
                        def _():
                            resolve(t + 2, k)
                    else:
                        @pl.when(i < DCH // 2 - 1)
                        def _():
                            wait_e(y)
                            rowgather(t + 1, y)
                            resolve(t + 2, k)

                    compute(t, k)
                return carry

            lax.fori_loop(0, DCH // 2, body, 0)
            pltpu.sync_copy(out_buf, out_hbm.at[pl.ds(s * DCH, DCH)])

    process(g_lo, gq_lo, lo_hbm, 0)
    process(g_hi, gq_hi, hi_hbm, 1)


_decode = pl.kernel(
    _decode_body,
    out_type=[jax.ShapeDtypeStruct((NCH_M, CHUNK // 16, 16),
                                   jnp.float32)] * 2,
    mesh=_MESH,
    scratch_types=[
        pltpu.VMEM((DCH, CHUNK), jnp.int32),
        pltpu.VMEM((CHUNK,), jnp.int32),
        pltpu.VMEM((CHUNK,), jnp.int32),
        pltpu.VMEM((CHUNK,), jnp.int32),
        pltpu.VMEM((CHUNK,), jnp.int32),
        pltpu.VMEM((CHUNK, HALF), jnp.float32),
        pltpu.VMEM((CHUNK, HALF), jnp.float32),
        pltpu.VMEM((CHUNK, HALF), jnp.float32),
        pltpu.VMEM((CHUNK, HALF), jnp.float32),
        pltpu.VMEM((DCH, CHUNK // 16, 16), jnp.float32),
        pltpu.VMEM((16, 32), jnp.float32),
        pltpu.SemaphoreType.DMA,
        pltpu.SemaphoreType.DMA,
        pltpu.SemaphoreType.DMA,
        pltpu.SemaphoreType.DMA,
    ],
)


def kernel(x, edge_sim, edge_rat, mask,
           W1_self, W1_sim, W1_rat, b1,
           W2_self, W2_sim, W2_rat, b2, Q):
    w1 = jnp.concatenate([_pad_w(W1_self), _pad_w(W1_sim), _pad_w(W1_rat)],
                         axis=1)
    w2 = jnp.concatenate([_pad_w(W2_self), _pad_w(W2_sim), _pad_w(W2_rat)],
                         axis=1)
    w2 = jnp.pad(w2, ((0, DP - D_H), (0, 0)))
    qp = jnp.pad(_pad_w(Q), ((0, DP - D_H), (0, 0)))
    b1p = jnp.pad(b1, (0, DP - D_H)).reshape(1, DP)
    b2p = jnp.pad(b2, (0, DP - D_H)).reshape(1, DP)
    xp = jnp.pad(x, ((0, NP - N), (0, 0)))
    sim_src = _pad_idx(edge_sim[0], NCH_E)
    sim_dst = _pad_idx(edge_sim[1], NCH_E)
    rat_src = edge_rat[0]
    rat_dst = edge_rat[1]
    rat_src2d = _pad_idx(rat_src, NCH_E)
    rat_dst2d = _pad_idx(rat_dst, NCH_E)
    n_mpad = M_PAD - M
    mask_p = jnp.concatenate(
        [mask, jnp.arange(n_mpad, dtype=jnp.int32) % 1024]
    ).reshape(NCH_M, CHUNK)

    xf_lo, xf_hi, xs_lo, xs_hi, xr_lo, xr_hi = _mm1(xp, w1)
    p_lo, p_hi = _segsum(xs_lo, xs_hi, xr_lo, xr_hi, xf_lo, xf_hi,
                         sim_src, sim_dst, rat_src2d, rat_dst2d)
    hf_lo, hf_hi, hs_lo, hs_hi, hr_lo, hr_hi = _mm2(p_lo, p_hi, b1p, w2)
    r_lo, r_hi = _segsum(hs_lo, hs_hi, hr_lo, hr_hi, hf_lo, hf_hi,
                         sim_src, sim_dst, rat_src2d, rat_dst2d)
    g_lo, g_hi, gq_lo, gq_hi = _mm3(r_lo, r_hi, b2p, qp)
    d_lo, d_hi = _decode(g_lo, g_hi, gq_lo, gq_hi, rat_src, rat_dst, mask_p)
    return _fin(d_lo.reshape(NCH_M, CHUNK),
                d_hi.reshape(NCH_M, CHUNK)).reshape(M_PAD)[:M]


# speedup vs baseline: 1.1045x; 1.1045x over previous
import jax
import jax.numpy as jnp
from jax import lax
from jax.experimental import pallas as pl
from jax.experimental.pallas import tpu as pltpu
from jax.experimental.pallas import tpu_sc as plsc

N = 10000
NP = 10240
E = 160000
D_IN = 256
D_H = 200
DP = 256
HALF = 128
M = 80000
MEAN_RATING = 3.5

NC = 2
NS = 16
NW = NC * NS
CHUNK = 128
SCH = 112
SNCH = 1536
STCH = SNCH // NS
M_PAD = 81920
NCH_M = M_PAD // CHUNK
DCH = NCH_M // NS
ROWS_PER_TILE = NP // NS
ZROWS = 16

_MESH = plsc.VectorSubcoreMesh(core_axis_name="c", subcore_axis_name="s")


def _pad_w(w):
    return jnp.pad(w, ((0, 0), (0, DP - D_H)))


def _pad_idx(idx1d):
    n_pad = SNCH * SCH - idx1d.shape[0]
    pad = N + (jnp.arange(n_pad, dtype=jnp.int32) % 240)
    return jnp.concatenate([idx1d, pad]).reshape(SNCH, SCH)


_RB = 1024


def _mm1_body(x_ref, w_ref, xfl_ref, xfh_ref, xsl_ref, xsh_ref,
              xrl_ref, xrh_ref):
    prod = jnp.dot(x_ref[...], w_ref[...], preferred_element_type=jnp.float32)
    xfl_ref[...] = prod[:, :HALF]
    xfh_ref[...] = prod[:, HALF:DP]
    xsl_ref[...] = prod[:, DP:DP + HALF]
    xsh_ref[...] = prod[:, DP + HALF:2 * DP]
    xrl_ref[...] = prod[:, 2 * DP:2 * DP + HALF]
    xrh_ref[...] = prod[:, 2 * DP + HALF:]


def _mm1(x, w1):
    return pl.pallas_call(
        _mm1_body,
        grid=(NP // _RB,),
        in_specs=[
            pl.BlockSpec((_RB, D_IN), lambda i: (i, 0)),
            pl.BlockSpec((D_IN, 3 * DP), lambda i: (0, 0)),
        ],
        out_specs=[pl.BlockSpec((_RB, HALF), lambda i: (i, 0))] * 6,
        out_shape=[jax.ShapeDtypeStruct((NP, HALF), jnp.float32)] * 6,
    )(x, w1)


def _mm2_body(pl_ref, ph_ref, b_ref, w_ref,
              hfl_ref, hfh_ref, hsl_ref, hsh_ref, hrl_ref, hrh_ref):
    agg = jnp.concatenate([pl_ref[...], ph_ref[...]], axis=1)
    h = jax.nn.relu(agg + b_ref[...])
    prod = jnp.dot(h, w_ref[...], preferred_element_type=jnp.float32)
    hfl_ref[...] = prod[:, :HALF]
    hfh_ref[...] = prod[:, HALF:DP]
    hsl_ref[...] = prod[:, DP:DP + HALF]
    hsh_ref[...] = prod[:, DP + HALF:2 * DP]
    hrl_ref[...] = prod[:, 2 * DP:2 * DP + HALF]
    hrh_ref[...] = prod[:, 2 * DP + HALF:]


def _mm2(p_lo, p_hi, b1, w2):
    return pl.pallas_call(
        _mm2_body,
        grid=(NP // _RB,),
        in_specs=[
            pl.BlockSpec((_RB, HALF), lambda i: (i, 0)),
            pl.BlockSpec((_RB, HALF), lambda i: (i, 0)),
            pl.BlockSpec((1, DP), lambda i: (0, 0)),
            pl.BlockSpec((DP, 3 * DP), lambda i: (0, 0)),
        ],
        out_specs=[pl.BlockSpec((_RB, HALF), lambda i: (i, 0))] * 6,
        out_shape=[jax.ShapeDtypeStruct((NP, HALF), jnp.float32)] * 6,
    )(p_lo, p_hi, b1, w2)


def _mm3_body(rl_ref, rh_ref, b_ref, q_ref,
              gl_ref, gh_ref, ql_ref, qh_ref):
    agg = jnp.concatenate([rl_ref[...], rh_ref[...]], axis=1)
    g = agg + b_ref[...]
    gl_ref[...] = g[:, :HALF]
    gh_ref[...] = g[:, HALF:]
    gq = jnp.dot(g, q_ref[...], preferred_element_type=jnp.float32)
    ql_ref[...] = gq[:, :HALF]
    qh_ref[...] = gq[:, HALF:]


def _mm3(r_lo, r_hi, b2, q):
    return pl.pallas_call(
        _mm3_body,
        grid=(NP // _RB,),
        in_specs=[
            pl.BlockSpec((_RB, HALF), lambda i: (i, 0)),
            pl.BlockSpec((_RB, HALF), lambda i: (i, 0)),
            pl.BlockSpec((1, DP), lambda i: (0, 0)),
            pl.BlockSpec((DP, DP), lambda i: (0, 0)),
        ],
        out_specs=[pl.BlockSpec((_RB, HALF), lambda i: (i, 0))] * 4,
        out_shape=[jax.ShapeDtypeStruct((NP, HALF), jnp.float32)] * 4,
    )(r_lo, r_hi, b2, q)


def _fin_body(lo_ref, hi_ref, o_ref):
    o_ref[...] = lo_ref[...] + hi_ref[...] + MEAN_RATING


def _fin(lo, hi):
    return pl.pallas_call(
        _fin_body,
        in_specs=[
            pl.BlockSpec((NCH_M, CHUNK), lambda: (0, 0)),
            pl.BlockSpec((NCH_M, CHUNK), lambda: (0, 0)),
        ],
        out_specs=pl.BlockSpec((NCH_M, CHUNK), lambda: (0, 0)),
        out_shape=jax.ShapeDtypeStruct((NCH_M, CHUNK), jnp.float32),
    )(lo, hi)


SBCH = 24


def _segsum_body(xs_lo, xs_hi, xr_lo, xr_hi, xf_lo, xf_hi, sim_src,
                 sim_dst, rat_src, rat_dst, plo_hbm, phi_hbm,
                 acc, idx_s, idx_d, r0, r1, r2, g0, g1, g2, s0, s1, s2):
    c = lax.axis_index("c")
    s = lax.axis_index("s")
    rows = [r0, r1, r2]
    semg = [g0, g1, g2]
    sems = [s0, s1, s2]

    base_row = s * ROWS_PER_TILE

    @pl.when(c == 0)
    def _():
        pltpu.sync_copy(xf_lo.at[pl.ds(base_row, ROWS_PER_TILE)],
                        acc.at[pl.ds(base_row, ROWS_PER_TILE)])

    @pl.when(c == 1)
    def _():
        pltpu.sync_copy(xf_hi.at[pl.ds(base_row, ROWS_PER_TILE)],
                        acc.at[pl.ds(base_row, ROWS_PER_TILE)])

    plsc.subcore_barrier()

    def process(table_hbm, src2d, dst2d, core_id):
        @pl.when(c == core_id)
        def _():
            def gather(t, k):
                pltpu.async_copy(table_hbm.at[idx_s.at[t]], rows[k], semg[k])

            def scat(t, k):
                pltpu.async_copy(rows[k], acc.at[idx_d.at[t]], sems[k],
                                 add=True)

            def wait_g(k):
                pltpu.make_async_copy(table_hbm.at[pl.ds(0, SCH)],
                                      rows[k], semg[k]).wait()

            def wait_s(k):
                pltpu.make_async_copy(rows[k], acc.at[pl.ds(0, SCH)],
                                      sems[k]).wait()

            def blk(b, carry):
                blk_row = s * STCH + b * SBCH
                pltpu.sync_copy(src2d.at[pl.ds(blk_row, SBCH)], idx_s)
                pltpu.sync_copy(dst2d.at[pl.ds(blk_row, SBCH)], idx_d)
                gather(0, 0)
                gather(1, 1)

                def body(i, carry2):
                    for k in (0, 1, 2):
                        t = 3 * i + k
                        wait_g(k)
                        scat(t, k)
                        if k == 0:
                            @pl.when(i > 0)
                            def _():
                                wait_s(2)

                            gather(t + 2, 2)
                        elif k == 1:
                            wait_s(0)

                            @pl.when(i < SBCH // 3 - 1)
                            def _():
                                gather(t + 2, 0)
                        else:
                            wait_s(1)

                            @pl.when(i < SBCH // 3 - 1)
                            def _():
                                gather(t + 2, 1)
                    return carry2

                lax.fori_loop(0, SBCH // 3, body, 0)
                wait_s(2)
                return carry

            lax.fori_loop(0, STCH // SBCH, blk, 0)

    process(xs_lo, sim_src, sim_dst, 0)
    process(xs_hi, sim_src, sim_dst, 1)
    process(xr_lo, rat_src, rat_dst, 0)
    process(xr_hi, rat_src, rat_dst, 1)
    plsc.subcore_barrier()

    def dump(out_hbm, core_id):
        @pl.when(c == core_id)
        def _():
            pltpu.sync_copy(acc.at[pl.ds(base_row, ROWS_PER_TILE)],
                            out_hbm.at[pl.ds(base_row, ROWS_PER_TILE)])

    dump(plo_hbm, 0)
    dump(phi_hbm, 1)


_segsum = pl.kernel(
    _segsum_body,
    out_type=[jax.ShapeDtypeStruct((NP, HALF), jnp.float32)] * 2,
    mesh=_MESH,
    scratch_types=[
        pltpu.VMEM_SHARED((NP, HALF), jnp.float32),
        pltpu.VMEM((SBCH, SCH), jnp.int32),
        pltpu.VMEM((SBCH, SCH), jnp.int32),
    ]
    + [pltpu.VMEM((SCH, HALF), jnp.float32)] * 3
    + [pltpu.SemaphoreType.DMA] * 6,
)


def _decode_body(g_lo, g_hi, gq_lo, gq_hi, rat_src, rat_dst, mask2d,
                 lo_hbm, hi_hbm,
                 mvec, e1a, e1b, e2a, e2b, ra0, ra1, rb0, rb1, out_buf,
                 sm, se0, se1, sr0, sr1):
    c = lax.axis_index("c")
    s = lax.axis_index("s")
    e1 = [e1a, e1b]
    e2 = [e2a, e2b]
    ra = [ra0, ra1]
    rb = [rb0, rb1]
    seme = [se0, se1]
    semr = [sr0, sr1]

    def process(gt, qt, out_hbm, core_id):
        @pl.when(c == core_id)
        def _():
            pltpu.sync_copy(mask2d.at[pl.ds(s * DCH, DCH)], mvec)

            def resolve(t, k):
                pltpu.async_copy(rat_src.at[mvec.at[t]], e1[k], seme[k])
                pltpu.async_copy(rat_dst.at[mvec.at[t]], e2[k], seme[k])

            def wait_e(k):
                pltpu.make_async_copy(rat_src.at[pl.ds(0, CHUNK)],
                                      e1[k], seme[k]).wait()
                pltpu.make_async_copy(rat_src.at[pl.ds(0, CHUNK)],
                                      e2[k], seme[k]).wait()

            def rowgather(t, k):
                pltpu.async_copy(gt.at[e1[k]], ra[k], semr[k])
                pltpu.async_copy(qt.at[e2[k]], rb[k], semr[k])

            def wait_r(k):
                pltpu.make_async_copy(gt.at[pl.ds(0, CHUNK)],
                                      ra[k], semr[k]).wait()
                pltpu.make_async_copy(gt.at[pl.ds(0, CHUNK)],
                                      rb[k], semr[k]).wait()

            def compute(t, k):
                lanes = lax.iota(jnp.int32, 16)
                msk = {h: (lanes & (2 * h - 1)) < h for h in (8, 4, 2, 1)}

                def gbody(gi, carry):
                    vecs = []
                    for l in range(16):
                        e = gi * 16 + l
                        prods = [ra[k][e, pl.ds(16 * kk, 16)]
                                 * rb[k][e, pl.ds(16 * kk, 16)]
                                 for kk in range(HALF // 16)]
                        while len(prods) > 1:
                            prods = [prods[2 * i] + prods[2 * i + 1]
                                     for i in range(len(prods) // 2)]
                        vecs.append(prods[0])
                    while len(vecs) > 1:
                        half = len(vecs) // 2
                        h = half
                        nxt = []
                        for i in range(half):
                            a, b = vecs[i], vecs[i + half]
                            sm[2 * i, pl.ds(0, 16)] = a
                            da = sm[2 * i, pl.ds(h, 16)]
                            sm[2 * i + 1, pl.ds(h, 16)] = b
                            ub = sm[2 * i + 1, pl.ds(0, 16)]
                            nxt.append(jnp.where(msk[h], a + da, b + ub))
                        vecs = nxt
                    out_buf[t, gi, :] = vecs[0]
                    return carry

                lax.fori_loop(0, CHUNK // 16, gbody, 0)

            resolve(0, 0)
            wait_e(0)
            rowgather(0, 0)
            resolve(1, 1)

            def body(i, carry):
                for k in (0, 1):
                    t = 2 * i + k
                    y = 1 - k
                    wait_r(k)
                    if k == 0:
                        wait_e(y)
                        rowgather(t + 1, y)

                        @pl.when(i < DCH // 2 - 1)
                        def _():
                            resolve(t + 2, k)
                    else:
                        @pl.when(i < DCH // 2 - 1)
                        def _():
                            wait_e(y)
                            rowgather(t + 1, y)
                            resolve(t + 2, k)

                    compute(t, k)
                return carry

            lax.fori_loop(0, DCH // 2, body, 0)
            pltpu.sync_copy(out_buf, out_hbm.at[pl.ds(s * DCH, DCH)])

    process(g_lo, gq_lo, lo_hbm, 0)
    process(g_hi, gq_hi, hi_hbm, 1)


_decode = pl.kernel(
    _decode_body,
    out_type=[jax.ShapeDtypeStruct((NCH_M, CHUNK // 16, 16),
                                   jnp.float32)] * 2,
    mesh=_MESH,
    scratch_types=[
        pltpu.VMEM((DCH, CHUNK), jnp.int32),
        pltpu.VMEM((CHUNK,), jnp.int32),
        pltpu.VMEM((CHUNK,), jnp.int32),
        pltpu.VMEM((CHUNK,), jnp.int32),
        pltpu.VMEM((CHUNK,), jnp.int32),
        pltpu.VMEM((CHUNK, HALF), jnp.float32),
        pltpu.VMEM((CHUNK, HALF), jnp.float32),
        pltpu.VMEM((CHUNK, HALF), jnp.float32),
        pltpu.VMEM((CHUNK, HALF), jnp.float32),
        pltpu.VMEM((DCH, CHUNK // 16, 16), jnp.float32),
        pltpu.VMEM((16, 32), jnp.float32),
        pltpu.SemaphoreType.DMA,
        pltpu.SemaphoreType.DMA,
        pltpu.SemaphoreType.DMA,
        pltpu.SemaphoreType.DMA,
    ],
)


def kernel(x, edge_sim, edge_rat, mask,
           W1_self, W1_sim, W1_rat, b1,
           W2_self, W2_sim, W2_rat, b2, Q):
    w1 = jnp.concatenate([_pad_w(W1_self), _pad_w(W1_sim), _pad_w(W1_rat)],
                         axis=1)
    w2 = jnp.concatenate([_pad_w(W2_self), _pad_w(W2_sim), _pad_w(W2_rat)],
                         axis=1)
    w2 = jnp.pad(w2, ((0, DP - D_H), (0, 0)))
    qp = jnp.pad(_pad_w(Q), ((0, DP - D_H), (0, 0)))
    b1p = jnp.pad(b1, (0, DP - D_H)).reshape(1, DP)
    b2p = jnp.pad(b2, (0, DP - D_H)).reshape(1, DP)
    xp = jnp.pad(x, ((0, NP - N), (0, 0)))
    sim_src = _pad_idx(edge_sim[0])
    sim_dst = _pad_idx(edge_sim[1])
    rat_src = edge_rat[0]
    rat_dst = edge_rat[1]
    rat_src2d = _pad_idx(rat_src)
    rat_dst2d = _pad_idx(rat_dst)
    n_mpad = M_PAD - M
    mask_p = jnp.concatenate(
        [mask, jnp.arange(n_mpad, dtype=jnp.int32) % 1024]
    ).reshape(NCH_M, CHUNK)

    xf_lo, xf_hi, xs_lo, xs_hi, xr_lo, xr_hi = _mm1(xp, w1)
    p_lo, p_hi = _segsum(xs_lo, xs_hi, xr_lo, xr_hi, xf_lo, xf_hi,
                         sim_src, sim_dst, rat_src2d, rat_dst2d)
    hf_lo, hf_hi, hs_lo, hs_hi, hr_lo, hr_hi = _mm2(p_lo, p_hi, b1p, w2)
    r_lo, r_hi = _segsum(hs_lo, hs_hi, hr_lo, hr_hi, hf_lo, hf_hi,
                         sim_src, sim_dst, rat_src2d, rat_dst2d)
    g_lo, g_hi, gq_lo, gq_hi = _mm3(r_lo, r_hi, b2p, qp)
    d_lo, d_hi = _decode(g_lo, g_hi, gq_lo, gq_hi, rat_src, rat_dst, mask_p)
    return _fin(d_lo.reshape(NCH_M, CHUNK),
                d_hi.reshape(NCH_M, CHUNK)).reshape(M_PAD)[:M]
